# baseline (speedup 1.0000x reference)
import functools

import jax
import jax.numpy as jnp
from jax.experimental import pallas as pl
from jax.experimental.pallas import tpu as pltpu


def _rank_lt(rows, k):
    sel = []
    for i, ci in enumerate(rows):
        rank = None
        for j, cj in enumerate(rows):
            if j == i:
                continue
            beats = (cj > ci) if j > i else (cj >= ci)
            b = beats.astype(jnp.float32)
            rank = b if rank is None else rank + b
        sel.append(rank < k)
    return sel


def _router_body(x_ref, gate_ref, comb_ref, *, n_group, topk_group, top_k):
    lt = jax.lax.dot_general(
        gate_ref[...], x_ref[...], (((1,), (1,)), ((), ())),
        preferred_element_type=jnp.float32)
    m = jnp.max(lt, axis=0, keepdims=True)
    p = jnp.exp(lt - m)
    p = p / jnp.sum(p, axis=0, keepdims=True)
    e_total = p.shape[0]
    per_g = e_total // n_group
    prows = [p[i:i + 1, :] for i in range(e_total)]
    grows = []
    for g in range(n_group):
        gc = prows[g * per_g]
        for r in range(1, per_g):
            gc = jnp.maximum(gc, prows[g * per_g + r])
        grows.append(gc)
    gsel = _rank_lt(grows, topk_group)
    trows = [jnp.where(gsel[i // per_g], prows[i], 0.0) for i in range(e_total)]
    esel = _rank_lt(trows, top_k)
    crows = [jnp.where(esel[i], trows[i], 0.0) for i in range(e_total)]
    comb_t = jnp.concatenate(crows, axis=0)
    comb_bt_e = comb_t.T
    for i in range(e_total):
        comb_ref[i] = comb_bt_e[:, i:i + 1]


def _moe_body(x_ref, comb_ref, wg_ref, wu_ref, w2_ref, out_ref,
              *, n_routed, bt, f):
    e = pl.program_id(0)
    t = pl.program_id(1)
    x32 = x_ref[...]

    w_col = jnp.where(e < n_routed, comb_ref[0], 1.0)

    xb = x32.astype(jnp.bfloat16)
    hg = jax.lax.dot_general(
        xb, wg_ref[0], (((1,), (1,)), ((), ())),
        preferred_element_type=jnp.float32)
    hu = jax.lax.dot_general(
        xb, wu_ref[0], (((1,), (1,)), ((), ())),
        preferred_element_type=jnp.float32)
    a = (hg * jax.nn.sigmoid(hg) * hu).astype(jnp.bfloat16)
    o = jax.lax.dot_general(
        a, w2_ref[0], (((1,), (1,)), ((), ())),
        preferred_element_type=jnp.float32)
    o = o * w_col
    rows = pl.ds(t * bt, bt)

    @pl.when(e == 0)
    def _():
        out_ref[rows, :] = o

    @pl.when(e > 0)
    def _():
        out_ref[rows, :] = out_ref[rows, :] + o


def kernel(x, gate_weight, w1, w2, shared_w1, shared_w2):
    T, D = x.shape
    E, F2, _ = w1.shape
    F = F2 // 2
    FS = shared_w2.shape[1]
    NP = FS // F
    n_group, topk_group, top_k = 4, 2, 2
    BT = min(512, T)

    comb = pl.pallas_call(
        functools.partial(_router_body, n_group=n_group,
                          topk_group=topk_group, top_k=top_k),
        grid=(T // BT,),
        in_specs=[
            pl.BlockSpec((BT, D), lambda t: (t, 0)),
            pl.BlockSpec((E, D), lambda t: (0, 0)),
        ],
        out_specs=pl.BlockSpec((E, BT, 1), lambda t: (0, t, 0)),
        out_shape=jax.ShapeDtypeStruct((E, T, 1), jnp.float32),
    )(x, gate_weight)

    sg = shared_w1[:FS]
    su = shared_w1[FS:]
    pw1 = jnp.stack([
        jnp.concatenate([sg[i * F:(i + 1) * F], su[i * F:(i + 1) * F]], axis=0)
        for i in range(NP)])
    pw2 = jnp.stack([shared_w2[:, i * F:(i + 1) * F] for i in range(NP)])
    W1 = jnp.concatenate([w1, pw1], axis=0).astype(jnp.bfloat16)
    W2 = jnp.concatenate([w2, pw2], axis=0).astype(jnp.bfloat16)
    NE = E + NP

    body = functools.partial(_moe_body, n_routed=E, bt=BT, f=F)

    return pl.pallas_call(
        body,
        grid=(NE, T // BT),
        in_specs=[
            pl.BlockSpec((BT, D), lambda e, t: (t, 0)),
            pl.BlockSpec((1, BT, 1), lambda e, t: (jnp.minimum(e, 7), t, 0)),
            pl.BlockSpec((1, F, D), lambda e, t: (e, 0, 0)),
            pl.BlockSpec((1, F, D), lambda e, t: (e, 1, 0)),
            pl.BlockSpec((1, D, F), lambda e, t: (e, 0, 0)),
        ],
        out_specs=pl.BlockSpec((T, D), lambda e, t: (0, 0)),
        out_shape=jax.ShapeDtypeStruct((T, D), jnp.float32),
        compiler_params=pltpu.CompilerParams(
            dimension_semantics=("arbitrary", "arbitrary"),
        ),
    )(x, comb, W1, W1, W2)


# speedup vs baseline: 1.0367x; 1.0367x over previous
import functools

import jax
import jax.numpy as jnp
from jax.experimental import pallas as pl
from jax.experimental.pallas import tpu as pltpu


def _rank_lt(rows, k):
    sel = []
    for i, ci in enumerate(rows):
        rank = None
        for j, cj in enumerate(rows):
            if j == i:
                continue
            beats = (cj > ci) if j > i else (cj >= ci)
            b = beats.astype(jnp.float32)
            rank = b if rank is None else rank + b
        sel.append(rank < k)
    return sel


def _router_body(x_ref, gate_ref, comb_ref, *, n_group, topk_group, top_k):
    lt = jax.lax.dot_general(
        gate_ref[...], x_ref[...], (((1,), (1,)), ((), ())),
        preferred_element_type=jnp.float32)
    m = jnp.max(lt, axis=0, keepdims=True)
    p = jnp.exp(lt - m)
    p = p / jnp.sum(p, axis=0, keepdims=True)
    e_total = p.shape[0]
    per_g = e_total // n_group
    prows = [p[i:i + 1, :] for i in range(e_total)]
    grows = []
    for g in range(n_group):
        gc = prows[g * per_g]
        for r in range(1, per_g):
            gc = jnp.maximum(gc, prows[g * per_g + r])
        grows.append(gc)
    gsel = _rank_lt(grows, topk_group)
    trows = [jnp.where(gsel[i // per_g], prows[i], 0.0) for i in range(e_total)]
    esel = _rank_lt(trows, top_k)
    crows = [jnp.where(esel[i], trows[i], 0.0) for i in range(e_total)]
    comb_t = jnp.concatenate(crows, axis=0)
    comb_bt_e = comb_t.T
    for i in range(e_total):
        comb_ref[i] = comb_bt_e[:, i:i + 1]


def _moe_body(x_ref, comb_ref, w1_ref, w2_ref, out_ref,
              *, n_routed, bt, f):
    e = pl.program_id(0)
    t = pl.program_id(1)
    x32 = x_ref[...]

    w_col = jnp.where(e < n_routed, comb_ref[0], 1.0)

    xb = x32.astype(jnp.bfloat16)
    h = jax.lax.dot_general(
        xb, w1_ref[0], (((1,), (1,)), ((), ())),
        preferred_element_type=jnp.float32)
    hg = h[:, :f]
    hu = h[:, f:]
    a = (hg * jax.nn.sigmoid(hg) * hu).astype(jnp.bfloat16)
    o = jax.lax.dot_general(
        a, w2_ref[0], (((1,), (1,)), ((), ())),
        preferred_element_type=jnp.float32)
    o = o * w_col
    rows = pl.ds(t * bt, bt)

    @pl.when(e == 0)
    def _():
        out_ref[rows, :] = o

    @pl.when(e > 0)
    def _():
        out_ref[rows, :] = out_ref[rows, :] + o


def kernel(x, gate_weight, w1, w2, shared_w1, shared_w2):
    T, D = x.shape
    E, F2, _ = w1.shape
    F = F2 // 2
    FS = shared_w2.shape[1]
    NP = FS // F
    n_group, topk_group, top_k = 4, 2, 2
    BT = min(512, T)

    comb = pl.pallas_call(
        functools.partial(_router_body, n_group=n_group,
                          topk_group=topk_group, top_k=top_k),
        grid=(T // BT,),
        in_specs=[
            pl.BlockSpec((BT, D), lambda t: (t, 0)),
            pl.BlockSpec((E, D), lambda t: (0, 0)),
        ],
        out_specs=pl.BlockSpec((E, BT, 1), lambda t: (0, t, 0)),
        out_shape=jax.ShapeDtypeStruct((E, T, 1), jnp.float32),
    )(x, gate_weight)

    sg = shared_w1[:FS]
    su = shared_w1[FS:]
    pw1 = jnp.stack([
        jnp.concatenate([sg[i * F:(i + 1) * F], su[i * F:(i + 1) * F]], axis=0)
        for i in range(NP)])
    pw2 = jnp.stack([shared_w2[:, i * F:(i + 1) * F] for i in range(NP)])
    W1 = jnp.concatenate([w1, pw1], axis=0).astype(jnp.bfloat16)
    W2 = jnp.concatenate([w2, pw2], axis=0).astype(jnp.bfloat16)
    NE = E + NP

    body = functools.partial(_moe_body, n_routed=E, bt=BT, f=F)

    return pl.pallas_call(
        body,
        grid=(NE, T // BT),
        in_specs=[
            pl.BlockSpec((BT, D), lambda e, t: (t, 0)),
            pl.BlockSpec((1, BT, 1), lambda e, t: (jnp.minimum(e, 7), t, 0)),
            pl.BlockSpec((1, F2, D), lambda e, t: (e, 0, 0)),
            pl.BlockSpec((1, D, F), lambda e, t: (e, 0, 0)),
        ],
        out_specs=pl.BlockSpec((T, D), lambda e, t: (0, 0)),
        out_shape=jax.ShapeDtypeStruct((T, D), jnp.float32),
        compiler_params=pltpu.CompilerParams(
            dimension_semantics=("arbitrary", "arbitrary"),
        ),
    )(x, comb, W1, W2)


# speedup vs baseline: 1.0764x; 1.0383x over previous
import functools

import jax
import jax.numpy as jnp
from jax import lax
from jax.experimental import pallas as pl
from jax.experimental.pallas import tpu as pltpu
from jax.experimental.pallas import tpu_sc as plsc

BLK = 256


def _rank_lt(rows, k):
    sel = []
    for i, ci in enumerate(rows):
        rank = None
        for j, cj in enumerate(rows):
            if j == i:
                continue
            beats = (cj > ci) if j > i else (cj >= ci)
            b = beats.astype(jnp.float32)
            rank = b if rank is None else rank + b
        sel.append(rank < k)
    return sel


def _router_body(x_ref, gate_ref, comb_ref, *, n_group, topk_group, top_k):
    lt = jax.lax.dot_general(
        gate_ref[...], x_ref[...], (((1,), (1,)), ((), ())),
        preferred_element_type=jnp.float32)
    m = jnp.max(lt, axis=0, keepdims=True)
    p = jnp.exp(lt - m)
    p = p / jnp.sum(p, axis=0, keepdims=True)
    e_total = p.shape[0]
    per_g = e_total // n_group
    prows = [p[i:i + 1, :] for i in range(e_total)]
    grows = []
    for g in range(n_group):
        gc = prows[g * per_g]
        for r in range(1, per_g):
            gc = jnp.maximum(gc, prows[g * per_g + r])
        grows.append(gc)
    gsel = _rank_lt(grows, topk_group)
    trows = [jnp.where(gsel[i // per_g], prows[i], 0.0) for i in range(e_total)]
    esel = _rank_lt(trows, top_k)
    crows = [jnp.where(esel[i], trows[i], 0.0) for i in range(e_total)]
    comb_ref[...] = jnp.concatenate(crows, axis=0)


def _planner_body(comb_ref, pos_ref, blk_ref, *, n_experts, t_tokens,
                  n_slots):
    E, T = n_experts, t_tokens
    NCH = T // 16
    mask = jnp.minimum(comb_ref[...] * 1e30, 1.0)
    jt = jax.lax.broadcasted_iota(jnp.int32, (T, T), 0)
    it2 = jax.lax.broadcasted_iota(jnp.int32, (T, T), 1)
    same = jnp.clip(1 - jnp.abs(jt // 16 - it2 // 16), 0, 1)
    lower = jnp.clip(it2 - jt, 0, 1)
    bds = (same * lower).astype(jnp.float32)
    rank = jax.lax.dot_general(
        mask, bds, (((1,), (0,)), ((), ())),
        preferred_element_type=jnp.float32, precision=jax.lax.Precision.HIGHEST)
    it = jax.lax.broadcasted_iota(jnp.int32, (T, NCH), 0)
    ic = jax.lax.broadcasted_iota(jnp.int32, (T, NCH), 1)
    dif = jnp.abs((it // 16) - ic)
    sel = jnp.clip(1 - dif, 0, 1).astype(jnp.float32)
    cnt = jax.lax.dot_general(
        mask, sel, (((1,), (0,)), ((), ())),
        preferred_element_type=jnp.float32, precision=jax.lax.Precision.HIGHEST)
    a = jax.lax.broadcasted_iota(jnp.int32, (NCH, NCH), 0)
    b = jax.lax.broadcasted_iota(jnp.int32, (NCH, NCH), 1)
    trin = jnp.clip(b - a, 0, 1).astype(jnp.float32)
    px = jax.lax.dot_general(
        cnt, trin, (((1,), (0,)), ((), ())),
        preferred_element_type=jnp.float32, precision=jax.lax.Precision.HIGHEST)
    tot = jax.lax.dot_general(
        cnt, jnp.ones((NCH, 1), jnp.float32), (((1,), (0,)), ((), ())),
        preferred_element_type=jnp.float32, precision=jax.lax.Precision.HIGHEST)
    padded = jnp.floor((tot + (BLK - 1)) / BLK) * BLK
    e1 = jax.lax.broadcasted_iota(jnp.int32, (E, E), 0)
    e2 = jax.lax.broadcasted_iota(jnp.int32, (E, E), 1)
    m8 = jnp.clip(e1 - e2, 0, 1).astype(jnp.float32)
    seg = jax.lax.dot_general(
        m8, padded, (((1,), (0,)), ((), ())),
        preferred_element_type=jnp.float32, precision=jax.lax.Precision.HIGHEST)
    pxtok = jax.lax.dot_general(
        px, sel, (((1,), (1,)), ((), ())),
        preferred_element_type=jnp.float32, precision=jax.lax.Precision.HIGHEST)
    pos_ref[...] = seg + pxtok + rank
    nblk = n_slots // BLK
    bi = jax.lax.broadcasted_iota(jnp.int32, (nblk, E), 0).astype(jnp.float32) * BLK
    segr = seg.reshape(1, E)
    cmp = jnp.clip(bi - segr + 1.0, 0.0, 1.0)
    nblkcnt = jax.lax.dot_general(
        cmp, jnp.ones((E, 1), jnp.float32), (((1,), (0,)), ((), ())),
        preferred_element_type=jnp.float32, precision=jax.lax.Precision.HIGHEST)
    blk_ref[...] = nblkcnt.astype(jnp.int32) - 1


def _dispatch_body(comb_hbm, pos_hbm, x_hbm, xs_hbm, pos0_hbm, pos1_hbm,
                   w0_hbm, w1_hbm, comb_v, posf_v, xrows_v, pos0_v, pos1_v,
                   w0_v, w1_v, sem, *, t_tokens, n_experts, n_workers,
                   n_cores):
    wid = lax.axis_index("s") * n_cores + lax.axis_index("c")
    tpw = t_tokens // n_workers
    tbase = wid * tpw
    my_first = wid * (tpw // 16)

    pltpu.sync_copy(comb_hbm, comb_v)
    pltpu.sync_copy(pos_hbm, posf_v)

    for c in range(tpw // 16):
        chunk = my_first + c
        pos0 = jnp.full((16,), -1, jnp.int32)
        pos1 = jnp.full((16,), -1, jnp.int32)
        wt0 = jnp.zeros((16,), jnp.float32)
        wt1 = jnp.zeros((16,), jnp.float32)
        for e in range(n_experts):
            cv = comb_v[e, pl.ds(chunk * 16, 16)]
            pf = posf_v[e, pl.ds(chunk * 16, 16)]
            mi = jnp.minimum(cv * 1e30, 1.0).astype(jnp.int32)
            posl = pf.astype(jnp.int32)
            asg = jnp.clip(pos0 + 1, 0, 1)
            take0 = mi * (1 - asg)
            take1 = mi * asg
            t0f = take0.astype(jnp.float32)
            t1f = take1.astype(jnp.float32)
            pos0 = take0 * posl + (1 - take0) * pos0
            wt0 = t0f * cv + (1.0 - t0f) * wt0
            pos1 = take1 * posl + (1 - take1) * pos1
            wt1 = t1f * cv + (1.0 - t1f) * wt1
        pos0_v[pl.ds(c * 16, 16)] = pos0
        pos1_v[pl.ds(c * 16, 16)] = pos1
        w0_v[pl.ds(c * 16, 16)] = wt0
        w1_v[pl.ds(c * 16, 16)] = wt1

    pltpu.sync_copy(pos0_v, pos0_hbm.at[pl.ds(tbase, tpw)])
    pltpu.sync_copy(pos1_v, pos1_hbm.at[pl.ds(tbase, tpw)])
    pltpu.sync_copy(w0_v, w0_hbm.at[pl.ds(tbase, tpw)])
    pltpu.sync_copy(w1_v, w1_hbm.at[pl.ds(tbase, tpw)])

    pltpu.sync_copy(x_hbm.at[pl.ds(tbase, tpw)], xrows_v)
    pltpu.async_copy(xrows_v, xs_hbm.at[pos0_v], sem).wait()
    pltpu.async_copy(xrows_v, xs_hbm.at[pos1_v], sem).wait()


def _group_mm_body(blk_ref, xs_ref, w1_ref, w2_ref, ys_ref, *, f):
    xb = xs_ref[...].astype(jnp.bfloat16)
    h = jax.lax.dot_general(
        xb, w1_ref[0], (((1,), (1,)), ((), ())),
        preferred_element_type=jnp.float32)
    hg = h[:, :f]
    hu = h[:, f:]
    a = (hg * jax.nn.sigmoid(hg) * hu).astype(jnp.bfloat16)
    ys_ref[...] = jax.lax.dot_general(
        a, w2_ref[0], (((1,), (1,)), ((), ())),
        preferred_element_type=jnp.float32)


def _shared_body(x_ref, w1_ref, w2_ref, out_ref, *, f, bt, n_pseudo):
    e = pl.program_id(0)
    t = pl.program_id(1)
    xb = x_ref[...].astype(jnp.bfloat16)
    h = jax.lax.dot_general(
        xb, w1_ref[0], (((1,), (1,)), ((), ())),
        preferred_element_type=jnp.float32)
    hg = h[:, :f]
    hu = h[:, f:]
    a = (hg * jax.nn.sigmoid(hg) * hu).astype(jnp.bfloat16)
    o = jax.lax.dot_general(
        a, w2_ref[0], (((1,), (1,)), ((), ())),
        preferred_element_type=jnp.float32)
    rows = pl.ds(t * bt, bt)

    @pl.when(e == 0)
    def _():
        out_ref[rows, :] = o

    @pl.when(e > 0)
    def _():
        out_ref[rows, :] = out_ref[rows, :] + o


def _combine_body(p0_ref, p1_ref, w0_ref, w1_ref, ys_ref, sh_ref, out_ref,
                  *, n_slots):
    bt = p0_ref.shape[1]
    slot = jax.lax.broadcasted_iota(jnp.int32, (n_slots, bt), 0)
    p0 = p0_ref[...]
    p1 = p1_ref[...]
    w0 = w0_ref[...]
    w1 = w1_ref[...]
    eq0 = jnp.clip(1 - jnp.abs(slot - p0), 0, 1).astype(jnp.float32)
    eq1 = jnp.clip(1 - jnp.abs(slot - p1), 0, 1).astype(jnp.float32)
    mt = (eq0 * w0 + eq1 * w1).astype(jnp.bfloat16)
    out_ref[...] = jax.lax.dot_general(
        mt, ys_ref[...], (((0,), (0,)), ((), ())),
        preferred_element_type=jnp.float32) + sh_ref[...]


def kernel(x, gate_weight, w1, w2, shared_w1, shared_w2):
    T, D = x.shape
    E, F2, _ = w1.shape
    F = F2 // 2
    FS = shared_w2.shape[1]
    NP = FS // F
    n_group, topk_group, top_k = 4, 2, 2
    BT = 512
    NSLOT = top_k * T + E * BLK - E
    NSLOT = ((NSLOT + BLK - 1) // BLK) * BLK
    NBLK = NSLOT // BLK

    info = plsc.get_sparse_core_info()
    NC, NS = info.num_cores, info.num_subcores
    NW = NC * NS

    comb_t = pl.pallas_call(
        functools.partial(_router_body, n_group=n_group,
                          topk_group=topk_group, top_k=top_k),
        grid=(T // BT,),
        in_specs=[
            pl.BlockSpec((BT, D), lambda t: (t, 0)),
            pl.BlockSpec((E, D), lambda t: (0, 0)),
        ],
        out_specs=pl.BlockSpec((E, BT), lambda t: (0, t)),
        out_shape=jax.ShapeDtypeStruct((E, T), jnp.float32),
    )(x, gate_weight)

    posf, blk2 = pl.pallas_call(
        functools.partial(_planner_body, n_experts=E, t_tokens=T,
                          n_slots=NSLOT),
        grid=(1,),
        in_specs=[pl.BlockSpec((E, T), lambda i: (0, 0))],
        out_specs=[
            pl.BlockSpec((E, T), lambda i: (0, 0)),
            pl.BlockSpec((NBLK, 1), lambda i: (0, 0)),
        ],
        out_shape=[
            jax.ShapeDtypeStruct((E, T), jnp.float32),
            jax.ShapeDtypeStruct((NBLK, 1), jnp.int32),
        ],
    )(comb_t)
    blk = blk2.reshape(NBLK)

    dispatch = pl.kernel(
        functools.partial(_dispatch_body, t_tokens=T, n_experts=E,
                          n_workers=NW, n_cores=NC),
        out_type=[
            jax.ShapeDtypeStruct((NSLOT, D), jnp.float32),
            jax.ShapeDtypeStruct((T,), jnp.int32),
            jax.ShapeDtypeStruct((T,), jnp.int32),
            jax.ShapeDtypeStruct((T,), jnp.float32),
            jax.ShapeDtypeStruct((T,), jnp.float32),
        ],
        mesh=plsc.VectorSubcoreMesh(core_axis_name="c", subcore_axis_name="s"),
        scratch_types=[
            pltpu.VMEM((E, T), jnp.float32),
            pltpu.VMEM((E, T), jnp.float32),
            pltpu.VMEM((T // NW, D), jnp.float32),
            pltpu.VMEM((T // NW,), jnp.int32),
            pltpu.VMEM((T // NW,), jnp.int32),
            pltpu.VMEM((T // NW,), jnp.float32),
            pltpu.VMEM((T // NW,), jnp.float32),
            pltpu.SemaphoreType.DMA,
        ],
    )
    xs, pos0, pos1, w0, w1v = dispatch(comb_t, posf, x)

    W1 = w1.astype(jnp.bfloat16)
    W2 = w2.astype(jnp.bfloat16)

    ys = pl.pallas_call(
        functools.partial(_group_mm_body, f=F),
        grid_spec=pltpu.PrefetchScalarGridSpec(
            num_scalar_prefetch=1,
            grid=(NBLK,),
            in_specs=[
                pl.BlockSpec((BLK, D), lambda b, blk_ref: (b, 0)),
                pl.BlockSpec((1, F2, D), lambda b, blk_ref: (blk_ref[b], 0, 0)),
                pl.BlockSpec((1, D, F), lambda b, blk_ref: (blk_ref[b], 0, 0)),
            ],
            out_specs=pl.BlockSpec((BLK, D), lambda b, blk_ref: (b, 0)),
        ),
        out_shape=jax.ShapeDtypeStruct((NSLOT, D), jnp.float32),
        compiler_params=pltpu.CompilerParams(
            dimension_semantics=("arbitrary",),
        ),
    )(blk, xs, W1, W2)

    sg = shared_w1[:FS]
    su = shared_w1[FS:]
    pw1 = jnp.stack([
        jnp.concatenate([sg[i * F:(i + 1) * F], su[i * F:(i + 1) * F]], axis=0)
        for i in range(NP)]).astype(jnp.bfloat16)
    pw2 = jnp.stack([shared_w2[:, i * F:(i + 1) * F]
                     for i in range(NP)]).astype(jnp.bfloat16)

    shared_out = pl.pallas_call(
        functools.partial(_shared_body, f=F, bt=BT, n_pseudo=NP),
        grid=(NP, T // BT),
        in_specs=[
            pl.BlockSpec((BT, D), lambda e, t: (t, 0)),
            pl.BlockSpec((1, F2, D), lambda e, t: (e, 0, 0)),
            pl.BlockSpec((1, D, F), lambda e, t: (e, 0, 0)),
        ],
        out_specs=pl.BlockSpec((T, D), lambda e, t: (0, 0)),
        out_shape=jax.ShapeDtypeStruct((T, D), jnp.float32),
        compiler_params=pltpu.CompilerParams(
            dimension_semantics=("arbitrary", "arbitrary"),
        ),
    )(x, pw1, pw2)

    out = pl.pallas_call(
        functools.partial(_combine_body, n_slots=NSLOT),
        grid=(T // BT,),
        in_specs=[
            pl.BlockSpec((1, BT), lambda t: (0, t)),
            pl.BlockSpec((1, BT), lambda t: (0, t)),
            pl.BlockSpec((1, BT), lambda t: (0, t)),
            pl.BlockSpec((1, BT), lambda t: (0, t)),
            pl.BlockSpec((NSLOT, D), lambda t: (0, 0)),
            pl.BlockSpec((BT, D), lambda t: (t, 0)),
        ],
        out_specs=pl.BlockSpec((BT, D), lambda t: (t, 0)),
        out_shape=jax.ShapeDtypeStruct((T, D), jnp.float32),
    )(pos0[None, :], pos1[None, :], w0[None, :], w1v[None, :],
      ys.astype(jnp.bfloat16), shared_out)
    return out
